# Initial kernel scaffold; baseline (speedup 1.0000x reference)
import jax
import jax.numpy as jnp
from jax.experimental import pallas as pl


def kernel(pokemon_ids, species_ability_ids, species_item_ids, species_move_ids, ability_ids, move_ids, item_ids, preparing_move_ids, raw_features, pokemon_static, pokemon_learn, ability_static, ability_learn, item_static, item_learn, move_static, move_learn):
    raise NotImplementedError("write your pallas kernel here")


# speedup vs baseline: 2.3433x; 2.3433x over previous
import jax
import jax.numpy as jnp
from jax import lax
from jax.experimental import pallas as pl
from jax.experimental.pallas import tpu as pltpu
from jax.experimental.pallas import tpu_sc as plsc

B = 1024
P = 12
S = B * P
NW = 32
SPW = S // NW
C = 16
NCH = SPW // C
IDXW = SPW * 17

_OFF_POK = 0
_OFF_SA = SPW
_OFF_SI = SPW * 4
_OFF_SM = SPW * 6
_OFF_ABIL = SPW * 10
_OFF_MV = SPW * 11
_OFF_ITEM = SPW * 15
_OFF_PREP = SPW * 16

_GATHERS = (
    (_OFF_POK, 1, 0, 0),
    (_OFF_SA, 3, 1, 1),
    (_OFF_SI, 2, 2, 2),
    (_OFF_SM, 4, 3, 3),
    (_OFF_ABIL, 1, 1, 4),
    (_OFF_MV, 4, 3, 5),
    (_OFF_ITEM, 1, 2, 6),
    (_OFF_PREP, 1, 3, 7),
)


def _sc_body(idx_hbm, pok_h, ab_h, it_h, mv_h,
             o_pok, o_sa, o_si, o_sm, o_abil, o_mv, o_item, o_prep,
             idx_v, b_pok, b_sa, b_si, b_sm, b_abil, b_mv, b_item, b_prep,
             sem):
    tables = (pok_h, ab_h, it_h, mv_h)
    bufs = (b_pok, b_sa, b_si, b_sm, b_abil, b_mv, b_item, b_prep)
    outs = (o_pok, o_sa, o_si, o_sm, o_abil, o_mv, o_item, o_prep)
    wid = lax.axis_index("s") * 2 + lax.axis_index("c")
    base = wid * SPW
    pltpu.sync_copy(idx_hbm.at[wid], idx_v)

    def chunk(i, carry):
        s0 = base + i * C
        handles = []
        for (off, k, t, b) in _GATHERS:
            io = pl.multiple_of(off + i * (k * C), 8)
            handles.append(pltpu.async_copy(
                tables[t].at[idx_v.at[pl.ds(io, k * C)]], bufs[b], sem))
        for h in handles:
            h.wait()
        for (off, k, t, b) in _GATHERS:
            oo = pl.multiple_of(k * s0, 8)
            pltpu.sync_copy(bufs[b], outs[b].at[pl.ds(oo, k * C)])
        return carry

    lax.fori_loop(0, NCH, chunk, 0)


@jax.jit
def _run(idx, pok_h, ab_h, it_h, mv_h):
    mesh = plsc.VectorSubcoreMesh(core_axis_name="c", subcore_axis_name="s")
    f = pl.kernel(
        _sc_body,
        out_type=(
            jax.ShapeDtypeStruct((S, 384), jnp.float32),
            jax.ShapeDtypeStruct((3 * S, 128), jnp.float32),
            jax.ShapeDtypeStruct((2 * S, 128), jnp.float32),
            jax.ShapeDtypeStruct((4 * S, 256), jnp.float32),
            jax.ShapeDtypeStruct((S, 128), jnp.float32),
            jax.ShapeDtypeStruct((4 * S, 256), jnp.float32),
            jax.ShapeDtypeStruct((S, 128), jnp.float32),
            jax.ShapeDtypeStruct((S, 256), jnp.float32),
        ),
        mesh=mesh,
        scratch_types=[
            pltpu.VMEM((IDXW,), jnp.int32),
            pltpu.VMEM((C, 384), jnp.float32),
            pltpu.VMEM((3 * C, 128), jnp.float32),
            pltpu.VMEM((2 * C, 128), jnp.float32),
            pltpu.VMEM((4 * C, 256), jnp.float32),
            pltpu.VMEM((C, 128), jnp.float32),
            pltpu.VMEM((4 * C, 256), jnp.float32),
            pltpu.VMEM((C, 128), jnp.float32),
            pltpu.VMEM((C, 256), jnp.float32),
            pltpu.SemaphoreType.DMA,
        ],
    )
    return f(idx, pok_h, ab_h, it_h, mv_h)


def kernel(pokemon_ids, species_ability_ids, species_item_ids,
           species_move_ids, ability_ids, move_ids, item_ids,
           preparing_move_ids, raw_features, pokemon_static, pokemon_learn,
           ability_static, ability_learn, item_static, item_learn,
           move_static, move_learn):
    wm = lambda x: x.reshape(S, -1).astype(jnp.int32).reshape(NW, -1)
    idx = jnp.concatenate([
        wm(pokemon_ids), wm(species_ability_ids), wm(species_item_ids),
        wm(species_move_ids), wm(ability_ids), wm(move_ids),
        wm(item_ids), wm(preparing_move_ids)], axis=1)
    def pad_to(x, w):
        return jnp.pad(x, ((0, 0), (0, w - x.shape[1])))
    pok_h = pad_to(jnp.concatenate([pokemon_static, pokemon_learn], axis=1), 384)
    ab_h = pad_to(jnp.concatenate([ability_static, ability_learn], axis=1), 128)
    it_h = pad_to(jnp.concatenate([item_static, item_learn], axis=1), 128)
    mv_h = pad_to(jnp.concatenate([move_static, move_learn], axis=1), 256)
    o = _run(idx, pok_h, ab_h, it_h, mv_h)
    out = jnp.concatenate([
        o[0][:, :291],
        o[1].reshape(S, 3, 128)[:, :, :51].reshape(S, 153),
        o[2].reshape(S, 2, 128)[:, :, :51].reshape(S, 102),
        o[3].reshape(S, 4, 256)[:, :, :154].reshape(S, 616),
        raw_features.reshape(S, 337).astype(jnp.float32),
        o[4][:, :51],
        o[5].reshape(S, 4, 256)[:, :, :154].reshape(S, 616),
        o[6][:, :51], o[7][:, :154]], axis=-1)
    return out.reshape(B, P * 2371)


# speedup vs baseline: 5.0310x; 2.1470x over previous
import jax
import jax.numpy as jnp
from jax import lax
from jax.experimental import pallas as pl
from jax.experimental.pallas import tpu as pltpu
from jax.experimental.pallas import tpu_sc as plsc

B = 1024
P = 12
S = B * P
NW = 32
SPW = S // NW
C = 16
NCH = SPW // C
IDXW = SPW * 17

_OFF_POK = 0
_OFF_SA = SPW
_OFF_SI = SPW * 4
_OFF_SM = SPW * 6
_OFF_ABIL = SPW * 10
_OFF_MV = SPW * 11
_OFF_ITEM = SPW * 15
_OFF_PREP = SPW * 16

_GATHERS = (
    (_OFF_POK, 1, 0, 0),
    (_OFF_SA, 3, 1, 1),
    (_OFF_SI, 2, 2, 2),
    (_OFF_SM, 4, 3, 3),
    (_OFF_ABIL, 1, 1, 4),
    (_OFF_MV, 4, 3, 5),
    (_OFF_ITEM, 1, 2, 6),
    (_OFF_PREP, 1, 3, 7),
)


def _sc_body(idx_hbm, pok_h, ab_h, it_h, mv_h,
             o_pok, o_sa, o_si, o_sm, o_abil, o_mv, o_item, o_prep,
             idx_v, b_pok, b_sa, b_si, b_sm, b_abil, b_mv, b_item, b_prep,
             sem):
    tables = (pok_h, ab_h, it_h, mv_h)
    bufs = (b_pok, b_sa, b_si, b_sm, b_abil, b_mv, b_item, b_prep)
    outs = (o_pok, o_sa, o_si, o_sm, o_abil, o_mv, o_item, o_prep)
    wid = lax.axis_index("s") * 2 + lax.axis_index("c")
    base = wid * SPW
    pltpu.sync_copy(idx_hbm.at[wid], idx_v)

    def chunk(i, carry):
        s0 = base + i * C
        handles = []
        for (off, k, t, b) in _GATHERS:
            io = pl.multiple_of(off + i * (k * C), 8)
            handles.append(pltpu.async_copy(
                tables[t].at[idx_v.at[pl.ds(io, k * C)]], bufs[b], sem))
        for h in handles:
            h.wait()
        for (off, k, t, b) in _GATHERS:
            oo = pl.multiple_of(k * s0, 8)
            pltpu.sync_copy(bufs[b], outs[b].at[pl.ds(oo, k * C)])
        return carry

    lax.fori_loop(0, NCH, chunk, 0)


def _asm_body(pok, sa, si, sm, abil, mv, item, prep, raw, out):
    pokv = pok[...].reshape(8, 12, 384)
    sav = sa[...].reshape(8, 12, 384)
    siv = si[...].reshape(8, 12, 256)
    smv = sm[...].reshape(8, 12, 1024)
    abv = abil[...].reshape(8, 12, 128)
    mvv = mv[...].reshape(8, 12, 1024)
    itv = item[...].reshape(8, 12, 128)
    prv = prep[...].reshape(8, 12, 256)
    rawv = raw[...].reshape(8, 12, 337)
    for p in range(12):
        seg = jnp.concatenate([
            pokv[:, p, :291],
            sav[:, p, 0:51], sav[:, p, 128:179], sav[:, p, 256:307],
            siv[:, p, 0:51], siv[:, p, 128:179],
            smv[:, p, 0:154], smv[:, p, 256:410],
            smv[:, p, 512:666], smv[:, p, 768:922],
            rawv[:, p, :],
            abv[:, p, :51],
            mvv[:, p, 0:154], mvv[:, p, 256:410],
            mvv[:, p, 512:666], mvv[:, p, 768:922],
            itv[:, p, :51], prv[:, p, :154]], axis=-1)
        out[:, p * 2371:(p + 1) * 2371] = seg


def _assemble(o, raw_flat):
    f = pl.pallas_call(
        _asm_body,
        grid=(B // 8,),
        in_specs=[
            pl.BlockSpec((96, 384), lambda i: (i, 0)),
            pl.BlockSpec((288, 128), lambda i: (i, 0)),
            pl.BlockSpec((192, 128), lambda i: (i, 0)),
            pl.BlockSpec((384, 256), lambda i: (i, 0)),
            pl.BlockSpec((96, 128), lambda i: (i, 0)),
            pl.BlockSpec((384, 256), lambda i: (i, 0)),
            pl.BlockSpec((96, 128), lambda i: (i, 0)),
            pl.BlockSpec((96, 256), lambda i: (i, 0)),
            pl.BlockSpec((96, 337), lambda i: (i, 0)),
        ],
        out_specs=pl.BlockSpec((8, P * 2371), lambda i: (i, 0)),
        out_shape=jax.ShapeDtypeStruct((B, P * 2371), jnp.float32),
    )
    return f(*o, raw_flat)


@jax.jit
def _run(idx, pok_h, ab_h, it_h, mv_h):
    mesh = plsc.VectorSubcoreMesh(core_axis_name="c", subcore_axis_name="s")
    f = pl.kernel(
        _sc_body,
        out_type=(
            jax.ShapeDtypeStruct((S, 384), jnp.float32),
            jax.ShapeDtypeStruct((3 * S, 128), jnp.float32),
            jax.ShapeDtypeStruct((2 * S, 128), jnp.float32),
            jax.ShapeDtypeStruct((4 * S, 256), jnp.float32),
            jax.ShapeDtypeStruct((S, 128), jnp.float32),
            jax.ShapeDtypeStruct((4 * S, 256), jnp.float32),
            jax.ShapeDtypeStruct((S, 128), jnp.float32),
            jax.ShapeDtypeStruct((S, 256), jnp.float32),
        ),
        mesh=mesh,
        scratch_types=[
            pltpu.VMEM((IDXW,), jnp.int32),
            pltpu.VMEM((C, 384), jnp.float32),
            pltpu.VMEM((3 * C, 128), jnp.float32),
            pltpu.VMEM((2 * C, 128), jnp.float32),
            pltpu.VMEM((4 * C, 256), jnp.float32),
            pltpu.VMEM((C, 128), jnp.float32),
            pltpu.VMEM((4 * C, 256), jnp.float32),
            pltpu.VMEM((C, 128), jnp.float32),
            pltpu.VMEM((C, 256), jnp.float32),
            pltpu.SemaphoreType.DMA,
        ],
    )
    return f(idx, pok_h, ab_h, it_h, mv_h)


def kernel(pokemon_ids, species_ability_ids, species_item_ids,
           species_move_ids, ability_ids, move_ids, item_ids,
           preparing_move_ids, raw_features, pokemon_static, pokemon_learn,
           ability_static, ability_learn, item_static, item_learn,
           move_static, move_learn):
    wm = lambda x: x.reshape(S, -1).astype(jnp.int32).reshape(NW, -1)
    idx = jnp.concatenate([
        wm(pokemon_ids), wm(species_ability_ids), wm(species_item_ids),
        wm(species_move_ids), wm(ability_ids), wm(move_ids),
        wm(item_ids), wm(preparing_move_ids)], axis=1)
    def pad_to(x, w):
        return jnp.pad(x, ((0, 0), (0, w - x.shape[1])))
    pok_h = pad_to(jnp.concatenate([pokemon_static, pokemon_learn], axis=1), 384)
    ab_h = pad_to(jnp.concatenate([ability_static, ability_learn], axis=1), 128)
    it_h = pad_to(jnp.concatenate([item_static, item_learn], axis=1), 128)
    mv_h = pad_to(jnp.concatenate([move_static, move_learn], axis=1), 256)
    o = _run(idx, pok_h, ab_h, it_h, mv_h)
    return _assemble(o, raw_features.reshape(S, 337).astype(jnp.float32))


# speedup vs baseline: 5.0445x; 1.0027x over previous
import jax
import jax.numpy as jnp
from jax import lax
from jax.experimental import pallas as pl
from jax.experimental.pallas import tpu as pltpu
from jax.experimental.pallas import tpu_sc as plsc

B = 1024
P = 12
S = B * P
NW = 32
SPW = S // NW
C = 8
NCH = SPW // C
IDXW = SPW * 17

_OFF_POK = 0
_OFF_SA = SPW
_OFF_SI = SPW * 4
_OFF_SM = SPW * 6
_OFF_ABIL = SPW * 10
_OFF_MV = SPW * 11
_OFF_ITEM = SPW * 15
_OFF_PREP = SPW * 16

_GATHERS = (
    (_OFF_POK, 1, 0, 0),
    (_OFF_SA, 3, 1, 1),
    (_OFF_SI, 2, 2, 2),
    (_OFF_SM, 4, 3, 3),
    (_OFF_ABIL, 1, 1, 4),
    (_OFF_MV, 4, 3, 5),
    (_OFF_ITEM, 1, 2, 6),
    (_OFF_PREP, 1, 3, 7),
)
_BUFSHAPES = ((C, 384), (3 * C, 128), (2 * C, 128), (4 * C, 256),
              (C, 128), (4 * C, 256), (C, 128), (C, 256))


def _sc_body(idx_hbm, pok_h, ab_h, it_h, mv_h,
             o_pok, o_sa, o_si, o_sm, o_abil, o_mv, o_item, o_prep,
             idx_v, bufs0, bufs1, sems):
    tables = (pok_h, ab_h, it_h, mv_h)
    bufsets = (bufs0, bufs1)
    outs = (o_pok, o_sa, o_si, o_sm, o_abil, o_mv, o_item, o_prep)
    wid = lax.axis_index("s") * 2 + lax.axis_index("c")
    base = wid * SPW
    pltpu.sync_copy(idx_hbm.at[wid], idx_v)
    gsem = (sems[0], sems[1])
    osem = (sems[2], sems[3])

    def issue_gathers(i, par):
        bufs = bufsets[par]
        for n, (off, k, t, b) in enumerate(_GATHERS):
            io = pl.multiple_of(off + i * (k * C), 8)
            pltpu.async_copy(
                tables[t].at[idx_v.at[pl.ds(io, k * C)]], bufs[b], gsem[par])

    def wait_gathers(par):
        bufs = bufsets[par]
        for (off, k, t, b) in _GATHERS:
            pltpu.make_async_copy(
                tables[t].at[pl.ds(0, k * C)], bufs[b], gsem[par]).wait()

    def issue_outs(i, par):
        bufs = bufsets[par]
        s0 = base + i * C
        for (off, k, t, b) in _GATHERS:
            oo = pl.multiple_of(k * s0, 8)
            pltpu.async_copy(bufs[b], outs[b].at[pl.ds(oo, k * C)], osem[par])

    def wait_outs(par):
        bufs = bufsets[par]
        for (off, k, t, b) in _GATHERS:
            pltpu.make_async_copy(
                bufs[b], outs[b].at[pl.ds(0, k * C)], osem[par]).wait()

    issue_gathers(0, 0)

    def step(i, carry):
        for par in (0, 1):
            j = i + par
            wait_gathers(par)
            issue_outs(j, par)
            nxt = 1 - par
            nj = j + 1

            @pl.when(nj < NCH)
            def _():
                @pl.when(nj >= 2)
                def _():
                    wait_outs(nxt)
                issue_gathers(nj, nxt)
        return carry

    lax.fori_loop(0, NCH // 2, lambda it, c: step(2 * it, c), 0, unroll=False)
    wait_outs(0)
    wait_outs(1)


def _asm_body(pok, sa, si, sm, abil, mv, item, prep, raw, out):
    pokv = pok[...].reshape(8, 12, 384)
    sav = sa[...].reshape(8, 12, 384)
    siv = si[...].reshape(8, 12, 256)
    smv = sm[...].reshape(8, 12, 1024)
    abv = abil[...].reshape(8, 12, 128)
    mvv = mv[...].reshape(8, 12, 1024)
    itv = item[...].reshape(8, 12, 128)
    prv = prep[...].reshape(8, 12, 256)
    rawv = raw[...]
    for p in range(12):
        seg = jnp.concatenate([
            pokv[:, p, :291],
            sav[:, p, 0:51], sav[:, p, 128:179], sav[:, p, 256:307],
            siv[:, p, 0:51], siv[:, p, 128:179],
            smv[:, p, 0:154], smv[:, p, 256:410],
            smv[:, p, 512:666], smv[:, p, 768:922],
            rawv[:, p, :],
            abv[:, p, :51],
            mvv[:, p, 0:154], mvv[:, p, 256:410],
            mvv[:, p, 512:666], mvv[:, p, 768:922],
            itv[:, p, :51], prv[:, p, :154]], axis=-1)
        out[:, p * 2371:(p + 1) * 2371] = seg


def _assemble(o, raw_flat):
    f = pl.pallas_call(
        _asm_body,
        grid=(B // 8,),
        in_specs=[
            pl.BlockSpec((96, 384), lambda i: (i, 0)),
            pl.BlockSpec((288, 128), lambda i: (i, 0)),
            pl.BlockSpec((192, 128), lambda i: (i, 0)),
            pl.BlockSpec((384, 256), lambda i: (i, 0)),
            pl.BlockSpec((96, 128), lambda i: (i, 0)),
            pl.BlockSpec((384, 256), lambda i: (i, 0)),
            pl.BlockSpec((96, 128), lambda i: (i, 0)),
            pl.BlockSpec((96, 256), lambda i: (i, 0)),
            pl.BlockSpec((8, 12, 337), lambda i: (i, 0, 0)),
        ],
        out_specs=pl.BlockSpec((8, P * 2371), lambda i: (i, 0)),
        out_shape=jax.ShapeDtypeStruct((B, P * 2371), jnp.float32),
    )
    return f(*o, raw_flat)


@jax.jit
def _run(idx, pok_h, ab_h, it_h, mv_h):
    mesh = plsc.VectorSubcoreMesh(core_axis_name="c", subcore_axis_name="s")
    f = pl.kernel(
        _sc_body,
        out_type=(
            jax.ShapeDtypeStruct((S, 384), jnp.float32),
            jax.ShapeDtypeStruct((3 * S, 128), jnp.float32),
            jax.ShapeDtypeStruct((2 * S, 128), jnp.float32),
            jax.ShapeDtypeStruct((4 * S, 256), jnp.float32),
            jax.ShapeDtypeStruct((S, 128), jnp.float32),
            jax.ShapeDtypeStruct((4 * S, 256), jnp.float32),
            jax.ShapeDtypeStruct((S, 128), jnp.float32),
            jax.ShapeDtypeStruct((S, 256), jnp.float32),
        ),
        mesh=mesh,
        scratch_types=[
            pltpu.VMEM((IDXW,), jnp.int32),
            tuple(pltpu.VMEM(s, jnp.float32) for s in _BUFSHAPES),
            tuple(pltpu.VMEM(s, jnp.float32) for s in _BUFSHAPES),
            (pltpu.SemaphoreType.DMA, pltpu.SemaphoreType.DMA,
             pltpu.SemaphoreType.DMA, pltpu.SemaphoreType.DMA),
        ],
    )
    return f(idx, pok_h, ab_h, it_h, mv_h)


def kernel(pokemon_ids, species_ability_ids, species_item_ids,
           species_move_ids, ability_ids, move_ids, item_ids,
           preparing_move_ids, raw_features, pokemon_static, pokemon_learn,
           ability_static, ability_learn, item_static, item_learn,
           move_static, move_learn):
    wm = lambda x: x.reshape(S, -1).astype(jnp.int32).reshape(NW, -1)
    idx = jnp.concatenate([
        wm(pokemon_ids), wm(species_ability_ids), wm(species_item_ids),
        wm(species_move_ids), wm(ability_ids), wm(move_ids),
        wm(item_ids), wm(preparing_move_ids)], axis=1)
    def pad_to(x, w):
        return jnp.pad(x, ((0, 0), (0, w - x.shape[1])))
    pok_h = pad_to(jnp.concatenate([pokemon_static, pokemon_learn], axis=1), 384)
    ab_h = pad_to(jnp.concatenate([ability_static, ability_learn], axis=1), 128)
    it_h = pad_to(jnp.concatenate([item_static, item_learn], axis=1), 128)
    mv_h = pad_to(jnp.concatenate([move_static, move_learn], axis=1), 256)
    o = _run(idx, pok_h, ab_h, it_h, mv_h)
    return _assemble(o, raw_features.astype(jnp.float32))


# speedup vs baseline: 5.3032x; 1.0513x over previous
import jax
import jax.numpy as jnp
from jax import lax
from jax.experimental import pallas as pl
from jax.experimental.pallas import tpu as pltpu
from jax.experimental.pallas import tpu_sc as plsc

B = 1024
P = 12
S = B * P
NW = 32
SPW = S // NW
C = 8
NCH = SPW // C
IDXW = SPW * 17

_OFF_POK = 0
_OFF_SA = SPW
_OFF_SI = SPW * 4
_OFF_SM = SPW * 6
_OFF_ABIL = SPW * 10
_OFF_MV = SPW * 11
_OFF_ITEM = SPW * 15
_OFF_PREP = SPW * 16

_GATHERS = (
    (_OFF_POK, 1, 0, 0),
    (_OFF_SA, 3, 1, 1),
    (_OFF_SI, 2, 2, 2),
    (_OFF_SM, 4, 3, 3),
    (_OFF_ABIL, 1, 1, 4),
    (_OFF_MV, 4, 3, 5),
    (_OFF_ITEM, 1, 2, 6),
    (_OFF_PREP, 1, 3, 7),
)
_BUFSHAPES = ((C, 384), (3 * C, 128), (2 * C, 128), (4 * C, 256),
              (C, 128), (4 * C, 256), (C, 128), (C, 256))


def _sc_body(idx_hbm, pok_h, ab_h, it_h, mv_h,
             o_pok, o_sa, o_si, o_sm, o_abil, o_mv, o_item, o_prep,
             idx_v, bufs0, bufs1, sems):
    tables = (pok_h, ab_h, it_h, mv_h)
    bufsets = (bufs0, bufs1)
    outs = (o_pok, o_sa, o_si, o_sm, o_abil, o_mv, o_item, o_prep)
    wid = lax.axis_index("s") * 2 + lax.axis_index("c")
    base = wid * SPW
    pltpu.sync_copy(idx_hbm.at[wid], idx_v)
    gsem = (sems[0], sems[1])
    osem = (sems[2], sems[3])

    def issue_gathers(i, par):
        bufs = bufsets[par]
        for n, (off, k, t, b) in enumerate(_GATHERS):
            io = pl.multiple_of(off + i * (k * C), 8)
            pltpu.async_copy(
                tables[t].at[idx_v.at[pl.ds(io, k * C)]], bufs[b], gsem[par])

    def wait_gathers(par):
        bufs = bufsets[par]
        for (off, k, t, b) in _GATHERS:
            pltpu.make_async_copy(
                tables[t].at[pl.ds(0, k * C)], bufs[b], gsem[par]).wait()

    def issue_outs(i, par):
        bufs = bufsets[par]
        s0 = base + i * C
        for (off, k, t, b) in _GATHERS:
            oo = pl.multiple_of(k * s0, 8)
            pltpu.async_copy(bufs[b], outs[b].at[pl.ds(oo, k * C)], osem[par])

    def wait_outs(par):
        bufs = bufsets[par]
        for (off, k, t, b) in _GATHERS:
            pltpu.make_async_copy(
                bufs[b], outs[b].at[pl.ds(0, k * C)], osem[par]).wait()

    issue_gathers(0, 0)

    def step(i, carry):
        for par in (0, 1):
            j = i + par
            wait_gathers(par)
            issue_outs(j, par)
            nxt = 1 - par
            nj = j + 1

            @pl.when(nj < NCH)
            def _():
                @pl.when(nj >= 2)
                def _():
                    wait_outs(nxt)
                issue_gathers(nj, nxt)
        return carry

    lax.fori_loop(0, NCH // 2, lambda it, c: step(2 * it, c), 0, unroll=False)
    wait_outs(0)
    wait_outs(1)


def _asm_body(pok, sa, si, sm, abil, mv, item, prep, raw, out):
    rawv = raw[...]
    for p in range(12):
        seg = jnp.concatenate([
            pok[p, 0][:, :291],
            sa[p, 0, 0][:, :51], sa[p, 0, 1][:, :51], sa[p, 0, 2][:, :51],
            si[p, 0, 0][:, :51], si[p, 0, 1][:, :51],
            sm[p, 0, 0][:, :154], sm[p, 0, 1][:, :154],
            sm[p, 0, 2][:, :154], sm[p, 0, 3][:, :154],
            rawv[:, p, :],
            abil[p, 0][:, :51],
            mv[p, 0, 0][:, :154], mv[p, 0, 1][:, :154],
            mv[p, 0, 2][:, :154], mv[p, 0, 3][:, :154],
            item[p, 0][:, :51], prep[p, 0][:, :154]], axis=-1)
        out[:, p * 2371:(p + 1) * 2371] = seg


def _assemble(o, raw):
    nc = B // C
    f = pl.pallas_call(
        _asm_body,
        grid=(nc,),
        in_specs=[
            pl.BlockSpec((P, 1, C, 384), lambda i: (0, i, 0, 0)),
            pl.BlockSpec((P, 1, 3, C, 128), lambda i: (0, i, 0, 0, 0)),
            pl.BlockSpec((P, 1, 2, C, 128), lambda i: (0, i, 0, 0, 0)),
            pl.BlockSpec((P, 1, 4, C, 256), lambda i: (0, i, 0, 0, 0)),
            pl.BlockSpec((P, 1, C, 128), lambda i: (0, i, 0, 0)),
            pl.BlockSpec((P, 1, 4, C, 256), lambda i: (0, i, 0, 0, 0)),
            pl.BlockSpec((P, 1, C, 128), lambda i: (0, i, 0, 0)),
            pl.BlockSpec((P, 1, C, 256), lambda i: (0, i, 0, 0)),
            pl.BlockSpec((C, P, 337), lambda i: (i, 0, 0)),
        ],
        out_specs=pl.BlockSpec((C, P * 2371), lambda i: (i, 0)),
        out_shape=jax.ShapeDtypeStruct((B, P * 2371), jnp.float32),
    )
    return f(o[0].reshape(P, nc, C, 384),
             o[1].reshape(P, nc, 3, C, 128),
             o[2].reshape(P, nc, 2, C, 128),
             o[3].reshape(P, nc, 4, C, 256),
             o[4].reshape(P, nc, C, 128),
             o[5].reshape(P, nc, 4, C, 256),
             o[6].reshape(P, nc, C, 128),
             o[7].reshape(P, nc, C, 256),
             raw)


@jax.jit
def _run(idx, pok_h, ab_h, it_h, mv_h):
    mesh = plsc.VectorSubcoreMesh(core_axis_name="c", subcore_axis_name="s")
    f = pl.kernel(
        _sc_body,
        out_type=(
            jax.ShapeDtypeStruct((S, 384), jnp.float32),
            jax.ShapeDtypeStruct((3 * S, 128), jnp.float32),
            jax.ShapeDtypeStruct((2 * S, 128), jnp.float32),
            jax.ShapeDtypeStruct((4 * S, 256), jnp.float32),
            jax.ShapeDtypeStruct((S, 128), jnp.float32),
            jax.ShapeDtypeStruct((4 * S, 256), jnp.float32),
            jax.ShapeDtypeStruct((S, 128), jnp.float32),
            jax.ShapeDtypeStruct((S, 256), jnp.float32),
        ),
        mesh=mesh,
        scratch_types=[
            pltpu.VMEM((IDXW,), jnp.int32),
            tuple(pltpu.VMEM(s, jnp.float32) for s in _BUFSHAPES),
            tuple(pltpu.VMEM(s, jnp.float32) for s in _BUFSHAPES),
            (pltpu.SemaphoreType.DMA, pltpu.SemaphoreType.DMA,
             pltpu.SemaphoreType.DMA, pltpu.SemaphoreType.DMA),
        ],
    )
    return f(idx, pok_h, ab_h, it_h, mv_h)


def kernel(pokemon_ids, species_ability_ids, species_item_ids,
           species_move_ids, ability_ids, move_ids, item_ids,
           preparing_move_ids, raw_features, pokemon_static, pokemon_learn,
           ability_static, ability_learn, item_static, item_learn,
           move_static, move_learn):
    def wm(x):
        x = x.reshape(B, P, -1).astype(jnp.int32)
        k = x.shape[2]
        x = x.transpose(1, 0, 2).reshape(P, B // C, C, k)
        return x.transpose(0, 1, 3, 2).reshape(NW, -1)
    idx = jnp.concatenate([
        wm(pokemon_ids), wm(species_ability_ids), wm(species_item_ids),
        wm(species_move_ids), wm(ability_ids), wm(move_ids),
        wm(item_ids), wm(preparing_move_ids)], axis=1)
    def pad_to(x, w):
        return jnp.pad(x, ((0, 0), (0, w - x.shape[1])))
    pok_h = pad_to(jnp.concatenate([pokemon_static, pokemon_learn], axis=1), 384)
    ab_h = pad_to(jnp.concatenate([ability_static, ability_learn], axis=1), 128)
    it_h = pad_to(jnp.concatenate([item_static, item_learn], axis=1), 128)
    mv_h = pad_to(jnp.concatenate([move_static, move_learn], axis=1), 256)
    o = _run(idx, pok_h, ab_h, it_h, mv_h)
    return _assemble(o, raw_features.astype(jnp.float32))


# speedup vs baseline: 5.7906x; 1.0919x over previous
import jax
import jax.numpy as jnp
from jax import lax
from jax.experimental import pallas as pl
from jax.experimental.pallas import tpu as pltpu
from jax.experimental.pallas import tpu_sc as plsc

B = 1024
P = 12
S = B * P
NW = 32
SPW = S // NW
C = 8
NCH = SPW // C
IDXW = SPW * 17

_OFF_POK = 0
_OFF_SA = SPW
_OFF_SI = SPW * 4
_OFF_SM = SPW * 6
_OFF_ABIL = SPW * 10
_OFF_MV = SPW * 11
_OFF_ITEM = SPW * 15
_OFF_PREP = SPW * 16

_GATHERS = (
    (_OFF_POK, 1, 0, 0),
    (_OFF_SA, 3, 1, 1),
    (_OFF_SI, 2, 2, 2),
    (_OFF_SM, 4, 3, 3),
    (_OFF_ABIL, 1, 1, 4),
    (_OFF_MV, 4, 3, 5),
    (_OFF_ITEM, 1, 2, 6),
    (_OFF_PREP, 1, 3, 7),
)
_BUFSHAPES = ((C, 384), (3 * C, 128), (2 * C, 128), (4 * C, 256),
              (C, 128), (4 * C, 256), (C, 128), (C, 256))


def _sc_body(idx_hbm, pok_h, ab_h, it_h, mv_h,
             o_pok, o_sa, o_si, o_sm, o_abil, o_mv, o_item, o_prep,
             idx_v, bufs0, bufs1, sems):
    tables = (pok_h, ab_h, it_h, mv_h)
    bufsets = (bufs0, bufs1)
    outs = (o_pok, o_sa, o_si, o_sm, o_abil, o_mv, o_item, o_prep)
    wid = lax.axis_index("s") * 2 + lax.axis_index("c")
    base = wid * SPW
    pltpu.sync_copy(idx_hbm.at[wid], idx_v)
    gsem = (sems[0], sems[1])
    osem = (sems[2], sems[3])

    def issue_gathers(i, par):
        bufs = bufsets[par]
        for n, (off, k, t, b) in enumerate(_GATHERS):
            io = pl.multiple_of(off + i * (k * C), 8)
            pltpu.async_copy(
                tables[t].at[idx_v.at[pl.ds(io, k * C)]], bufs[b], gsem[par])

    def wait_gathers(par):
        bufs = bufsets[par]
        for (off, k, t, b) in _GATHERS:
            pltpu.make_async_copy(
                tables[t].at[pl.ds(0, k * C)], bufs[b], gsem[par]).wait()

    def issue_outs(i, par):
        bufs = bufsets[par]
        s0 = base + i * C
        for (off, k, t, b) in _GATHERS:
            oo = pl.multiple_of(k * s0, 8)
            pltpu.async_copy(bufs[b], outs[b].at[pl.ds(oo, k * C)], osem[par])

    def wait_outs(par):
        bufs = bufsets[par]
        for (off, k, t, b) in _GATHERS:
            pltpu.make_async_copy(
                bufs[b], outs[b].at[pl.ds(0, k * C)], osem[par]).wait()

    issue_gathers(0, 0)

    def step(i, carry):
        for par in (0, 1):
            j = i + par
            wait_gathers(par)
            issue_outs(j, par)
            nxt = 1 - par
            nj = j + 1

            @pl.when(nj < NCH)
            def _():
                @pl.when(nj >= 2)
                def _():
                    wait_outs(nxt)
                issue_gathers(nj, nxt)
        return carry

    lax.fori_loop(0, NCH // 2, lambda it, c: step(2 * it, c), 0, unroll=False)
    wait_outs(0)
    wait_outs(1)


_Q = 4


def _asm_body(pok, sa, si, sm, abil, mv, item, prep, raw, out):
    rawv = raw[...]
    for q in range(_Q):
        rows = pl.ds(q * C, C)
        for p in range(12):
            seg = jnp.concatenate([
                pok[p, q][:, :291],
                sa[p, q, 0][:, :51], sa[p, q, 1][:, :51], sa[p, q, 2][:, :51],
                si[p, q, 0][:, :51], si[p, q, 1][:, :51],
                sm[p, q, 0][:, :154], sm[p, q, 1][:, :154],
                sm[p, q, 2][:, :154], sm[p, q, 3][:, :154],
                rawv[q * C:(q + 1) * C, p, :],
                abil[p, q][:, :51],
                mv[p, q, 0][:, :154], mv[p, q, 1][:, :154],
                mv[p, q, 2][:, :154], mv[p, q, 3][:, :154],
                item[p, q][:, :51], prep[p, q][:, :154]], axis=-1)
            out[rows, p * 2371:(p + 1) * 2371] = seg


def _assemble(o, raw):
    nc = B // C
    f = pl.pallas_call(
        _asm_body,
        grid=(nc // _Q,),
        in_specs=[
            pl.BlockSpec((P, _Q, C, 384), lambda i: (0, i, 0, 0)),
            pl.BlockSpec((P, _Q, 3, C, 128), lambda i: (0, i, 0, 0, 0)),
            pl.BlockSpec((P, _Q, 2, C, 128), lambda i: (0, i, 0, 0, 0)),
            pl.BlockSpec((P, _Q, 4, C, 256), lambda i: (0, i, 0, 0, 0)),
            pl.BlockSpec((P, _Q, C, 128), lambda i: (0, i, 0, 0)),
            pl.BlockSpec((P, _Q, 4, C, 256), lambda i: (0, i, 0, 0, 0)),
            pl.BlockSpec((P, _Q, C, 128), lambda i: (0, i, 0, 0)),
            pl.BlockSpec((P, _Q, C, 256), lambda i: (0, i, 0, 0)),
            pl.BlockSpec((_Q * C, P, 337), lambda i: (i, 0, 0)),
        ],
        out_specs=pl.BlockSpec((_Q * C, P * 2371), lambda i: (i, 0)),
        out_shape=jax.ShapeDtypeStruct((B, P * 2371), jnp.float32),
    )
    return f(o[0].reshape(P, nc, C, 384),
             o[1].reshape(P, nc, 3, C, 128),
             o[2].reshape(P, nc, 2, C, 128),
             o[3].reshape(P, nc, 4, C, 256),
             o[4].reshape(P, nc, C, 128),
             o[5].reshape(P, nc, 4, C, 256),
             o[6].reshape(P, nc, C, 128),
             o[7].reshape(P, nc, C, 256),
             raw)


@jax.jit
def _run(idx, pok_h, ab_h, it_h, mv_h):
    mesh = plsc.VectorSubcoreMesh(core_axis_name="c", subcore_axis_name="s")
    f = pl.kernel(
        _sc_body,
        out_type=(
            jax.ShapeDtypeStruct((S, 384), jnp.float32),
            jax.ShapeDtypeStruct((3 * S, 128), jnp.float32),
            jax.ShapeDtypeStruct((2 * S, 128), jnp.float32),
            jax.ShapeDtypeStruct((4 * S, 256), jnp.float32),
            jax.ShapeDtypeStruct((S, 128), jnp.float32),
            jax.ShapeDtypeStruct((4 * S, 256), jnp.float32),
            jax.ShapeDtypeStruct((S, 128), jnp.float32),
            jax.ShapeDtypeStruct((S, 256), jnp.float32),
        ),
        mesh=mesh,
        scratch_types=[
            pltpu.VMEM((IDXW,), jnp.int32),
            tuple(pltpu.VMEM(s, jnp.float32) for s in _BUFSHAPES),
            tuple(pltpu.VMEM(s, jnp.float32) for s in _BUFSHAPES),
            (pltpu.SemaphoreType.DMA, pltpu.SemaphoreType.DMA,
             pltpu.SemaphoreType.DMA, pltpu.SemaphoreType.DMA),
        ],
    )
    return f(idx, pok_h, ab_h, it_h, mv_h)


def kernel(pokemon_ids, species_ability_ids, species_item_ids,
           species_move_ids, ability_ids, move_ids, item_ids,
           preparing_move_ids, raw_features, pokemon_static, pokemon_learn,
           ability_static, ability_learn, item_static, item_learn,
           move_static, move_learn):
    def wm(x):
        x = x.reshape(B, P, -1).astype(jnp.int32)
        k = x.shape[2]
        x = x.transpose(1, 0, 2).reshape(P, B // C, C, k)
        return x.transpose(0, 1, 3, 2).reshape(NW, -1)
    idx = jnp.concatenate([
        wm(pokemon_ids), wm(species_ability_ids), wm(species_item_ids),
        wm(species_move_ids), wm(ability_ids), wm(move_ids),
        wm(item_ids), wm(preparing_move_ids)], axis=1)
    def pad_to(x, w):
        return jnp.pad(x, ((0, 0), (0, w - x.shape[1])))
    pok_h = pad_to(jnp.concatenate([pokemon_static, pokemon_learn], axis=1), 384)
    ab_h = pad_to(jnp.concatenate([ability_static, ability_learn], axis=1), 128)
    it_h = pad_to(jnp.concatenate([item_static, item_learn], axis=1), 128)
    mv_h = pad_to(jnp.concatenate([move_static, move_learn], axis=1), 256)
    o = _run(idx, pok_h, ab_h, it_h, mv_h)
    return _assemble(o, raw_features.astype(jnp.float32))
